# baseline (speedup 1.0000x reference)
import jax
import jax.numpy as jnp
from jax import lax
from jax.experimental import pallas as pl
from jax.experimental.pallas import tpu as pltpu

EPS = 1e-8
TOPK = 12


def _p1_body(th_ref, fp_ref, bp_ref, q_ref, pf_ref, pb_ref, af_ref, ab_ref,
             cnt_ref):
    h = pl.program_id(1)
    x = q_ref[0]
    fp = fp_ref[0]
    bp = bp_ref[0]
    tf = th_ref[0]
    tb = th_ref[1]
    qn = jnp.maximum(jnp.sqrt(jnp.sum(x * x, axis=0, keepdims=True)), EPS)
    nf = jnp.maximum(jnp.sqrt(jnp.sum(fp * fp)), EPS)
    nb = jnp.maximum(jnp.sqrt(jnp.sum(bp * bp)), EPS)
    sf = jnp.sum(x * fp, axis=0, keepdims=True) / (qn * nf)
    sb = jnp.sum(x * bp, axis=0, keepdims=True) / (qn * nb)
    z = 10.0 * (sf - sb)
    pf = 1.0 / (1.0 + jnp.exp(-z))
    pb = 1.0 / (1.0 + jnp.exp(z))
    pf_ref[0, 0] = pf
    pb_ref[0, 0] = pb
    mf = (pf > tf).astype(jnp.float32)
    mb = (pb > tb).astype(jnp.float32)
    sumf = jnp.sum(x * mf, axis=1, keepdims=True)[None]
    sumb = jnp.sum(x * mb, axis=1, keepdims=True)[None]
    lane = lax.broadcasted_iota(jnp.int32, (1, 1, 8), 2)
    crow = jnp.where(lane == 0, jnp.sum(mf),
                     jnp.where(lane == 1, jnp.sum(mb), 0.0))

    @pl.when(h == 0)
    def _():
        af_ref[...] = sumf
        ab_ref[...] = sumb
        cnt_ref[...] = crow

    @pl.when(h != 0)
    def _():
        af_ref[...] += sumf
        ab_ref[...] += sumb
        cnt_ref[...] += crow


def _tk1_body(pf_ref, pb_ref, wf_ref, wb_ref):
    hw = pf_ref.shape[-1]
    iota = lax.broadcasted_iota(jnp.int32, (1, hw), 1)

    def topw(p):
        x = p
        w = jnp.zeros_like(p)
        for _ in range(TOPK):
            m = jnp.max(x)
            fi = jnp.min(jnp.where(x == m, iota, hw))
            hit = iota == fi
            w = w + hit.astype(jnp.float32)
            x = jnp.where(hit, -jnp.inf, x)
        return w

    wf_ref[0] = topw(pf_ref[0])
    wb_ref[0] = topw(pb_ref[0])


def _tk2_body(q_ref, wf_ref, wb_ref, tf_ref, tb_ref):
    h = pl.program_id(1)
    x = q_ref[0]
    wf = wf_ref[0, 0]
    wb = wb_ref[0, 0]
    sumf = jnp.sum(x * wf, axis=1, keepdims=True)[None]
    sumb = jnp.sum(x * wb, axis=1, keepdims=True)[None]

    @pl.when(h == 0)
    def _():
        tf_ref[...] = sumf
        tb_ref[...] = sumb

    @pl.when(h != 0)
    def _():
        tf_ref[...] += sumf
        tb_ref[...] += sumb


def _p2_body(fgp_ref, bgp_ref, q_ref, a_ref, d_ref, mm_ref):
    b = pl.program_id(0)
    h = pl.program_id(1)
    x = q_ref[0]
    fg = fgp_ref[0]
    bg = bgp_ref[0]
    qn = jnp.maximum(jnp.sqrt(jnp.sum(x * x, axis=0, keepdims=True)), EPS)
    nf = jnp.maximum(jnp.sqrt(jnp.sum(fg * fg)), EPS)
    nb = jnp.maximum(jnp.sqrt(jnp.sum(bg * bg)), EPS)
    a = jnp.sum(x * fg, axis=0, keepdims=True) / (qn * nf)
    d = jnp.sum(x * bg, axis=0, keepdims=True) / (qn * nb)
    a_ref[0, 0] = a
    d_ref[0, 0] = d
    lane = lax.broadcasted_iota(jnp.int32, (1, 8), 1)
    row = jnp.where(lane == 0, jnp.min(a),
                    jnp.where(lane == 1, jnp.max(a),
                              jnp.where(lane == 2, jnp.min(d),
                                        jnp.where(lane == 3, jnp.max(d),
                                                  0.0))))

    @pl.when(jnp.logical_and(b == 0, h == 0))
    def _():
        mm_ref[...] = row

    @pl.when(jnp.logical_or(b != 0, h != 0))
    def _():
        cur = mm_ref[...]
        minl = jnp.logical_or(lane == 0, lane == 2)
        mm_ref[...] = jnp.where(minl, jnp.minimum(cur, row),
                                jnp.maximum(cur, row))


def _p3_body(mm_ref, q_ref, a_ref, d_ref, o_ref):
    x = q_ref[0]
    a = a_ref[0, 0]
    d = d_ref[0, 0]
    an = (a - mm_ref[0]) / (mm_ref[1] - mm_ref[0])
    dn = (d - mm_ref[2]) / (mm_ref[3] - mm_ref[2])
    o_ref[0] = x * (an + (1.0 - dn))


def kernel(supp_fp, supp_bp, query_fea, tau):
    bs, C, H, W = query_fea.shape
    hw = H * W
    T = 2048
    nblk = hw // T
    f32 = jnp.float32
    q = query_fea.reshape(bs, C, hw)
    fp = supp_fp.reshape(bs, C, 1)
    bp = supp_bp.reshape(bs, C, 1)
    ft = jax.nn.sigmoid(tau)
    th = jnp.stack([ft, 1.0 - ft]).astype(f32)

    pf, pb, af, ab, cnt = pl.pallas_call(
        _p1_body,
        grid=(bs, nblk),
        in_specs=[
            pl.BlockSpec(memory_space=pltpu.SMEM),
            pl.BlockSpec((1, C, 1), lambda b, h: (b, 0, 0)),
            pl.BlockSpec((1, C, 1), lambda b, h: (b, 0, 0)),
            pl.BlockSpec((1, C, T), lambda b, h: (b, 0, h)),
        ],
        out_specs=[
            pl.BlockSpec((1, 1, 1, T), lambda b, h: (b, h, 0, 0)),
            pl.BlockSpec((1, 1, 1, T), lambda b, h: (b, h, 0, 0)),
            pl.BlockSpec((1, C, 1), lambda b, h: (b, 0, 0)),
            pl.BlockSpec((1, C, 1), lambda b, h: (b, 0, 0)),
            pl.BlockSpec((1, 1, 8), lambda b, h: (b, 0, 0)),
        ],
        out_shape=[
            jax.ShapeDtypeStruct((bs, nblk, 1, T), f32),
            jax.ShapeDtypeStruct((bs, nblk, 1, T), f32),
            jax.ShapeDtypeStruct((bs, C, 1), f32),
            jax.ShapeDtypeStruct((bs, C, 1), f32),
            jax.ShapeDtypeStruct((bs, 1, 8), f32),
        ],
    )(th, fp, bp, q)

    cf = cnt[:, 0, 0]
    cb = cnt[:, 0, 1]
    need = jnp.logical_or(jnp.any(cf == 0), jnp.any(cb == 0))

    def topk_fn(_):
        wf, wb = pl.pallas_call(
            _tk1_body,
            grid=(bs,),
            in_specs=[
                pl.BlockSpec((1, 1, hw), lambda b: (b, 0, 0)),
                pl.BlockSpec((1, 1, hw), lambda b: (b, 0, 0)),
            ],
            out_specs=[
                pl.BlockSpec((1, 1, hw), lambda b: (b, 0, 0)),
                pl.BlockSpec((1, 1, hw), lambda b: (b, 0, 0)),
            ],
            out_shape=[
                jax.ShapeDtypeStruct((bs, 1, hw), f32),
                jax.ShapeDtypeStruct((bs, 1, hw), f32),
            ],
        )(pf.reshape(bs, 1, hw), pb.reshape(bs, 1, hw))
        tf, tb = pl.pallas_call(
            _tk2_body,
            grid=(bs, nblk),
            in_specs=[
                pl.BlockSpec((1, C, T), lambda b, h: (b, 0, h)),
                pl.BlockSpec((1, 1, 1, T), lambda b, h: (b, h, 0, 0)),
                pl.BlockSpec((1, 1, 1, T), lambda b, h: (b, h, 0, 0)),
            ],
            out_specs=[
                pl.BlockSpec((1, C, 1), lambda b, h: (b, 0, 0)),
                pl.BlockSpec((1, C, 1), lambda b, h: (b, 0, 0)),
            ],
            out_shape=[
                jax.ShapeDtypeStruct((bs, C, 1), f32),
                jax.ShapeDtypeStruct((bs, C, 1), f32),
            ],
        )(q, wf.reshape(bs, nblk, 1, T), wb.reshape(bs, nblk, 1, T))
        return tf[:, :, 0] / TOPK, tb[:, :, 0] / TOPK

    def zeros_fn(_):
        return jnp.zeros((bs, C), f32), jnp.zeros((bs, C), f32)

    tkf, tkb = lax.cond(need, topk_fn, zeros_fn, None)

    mf = af[:, :, 0] / jnp.maximum(cf, 1.0)[:, None]
    mb = ab[:, :, 0] / jnp.maximum(cb, 1.0)[:, None]
    fgp = jnp.where((cf > 0)[:, None], mf, tkf)
    bgp = jnp.where((cb > 0)[:, None], mb, tkb)

    a, d, mm = pl.pallas_call(
        _p2_body,
        grid=(bs, nblk),
        in_specs=[
            pl.BlockSpec((1, C, 1), lambda b, h: (b, 0, 0)),
            pl.BlockSpec((1, C, 1), lambda b, h: (b, 0, 0)),
            pl.BlockSpec((1, C, T), lambda b, h: (b, 0, h)),
        ],
        out_specs=[
            pl.BlockSpec((1, 1, 1, T), lambda b, h: (b, h, 0, 0)),
            pl.BlockSpec((1, 1, 1, T), lambda b, h: (b, h, 0, 0)),
            pl.BlockSpec((1, 8), lambda b, h: (0, 0)),
        ],
        out_shape=[
            jax.ShapeDtypeStruct((bs, nblk, 1, T), f32),
            jax.ShapeDtypeStruct((bs, nblk, 1, T), f32),
            jax.ShapeDtypeStruct((1, 8), f32),
        ],
    )(fgp.reshape(bs, C, 1), bgp.reshape(bs, C, 1), q)

    qo = pl.pallas_call(
        _p3_body,
        grid=(bs, nblk),
        in_specs=[
            pl.BlockSpec(memory_space=pltpu.SMEM),
            pl.BlockSpec((1, C, T), lambda b, h: (b, 0, h)),
            pl.BlockSpec((1, 1, 1, T), lambda b, h: (b, h, 0, 0)),
            pl.BlockSpec((1, 1, 1, T), lambda b, h: (b, h, 0, 0)),
        ],
        out_specs=pl.BlockSpec((1, C, T), lambda b, h: (b, 0, h)),
        out_shape=jax.ShapeDtypeStruct((bs, C, hw), f32),
    )(mm.reshape(8), q, a, d)

    return (qo.reshape(bs, C, H, W), fgp.reshape(bs, C, 1, 1),
            bgp.reshape(bs, C, 1, 1))


# speedup vs baseline: 2.0343x; 2.0343x over previous
import jax
import jax.numpy as jnp
from jax import lax
from jax.experimental import pallas as pl
from jax.experimental.pallas import tpu as pltpu

EPS = 1e-8
TOPK = 12


def _p1_body(th_ref, fp_ref, bp_ref, q_ref, pf_ref, pb_ref, af_ref, ab_ref,
             cnt_ref):
    h = pl.program_id(1)
    x = q_ref[0]
    fp = fp_ref[0]
    bp = bp_ref[0]
    tf = th_ref[0]
    tb = th_ref[1]
    qn = jnp.maximum(jnp.sqrt(jnp.sum(x * x, axis=0)), EPS)
    nf = jnp.maximum(jnp.sqrt(jnp.sum(fp * fp)), EPS)
    nb = jnp.maximum(jnp.sqrt(jnp.sum(bp * bp)), EPS)
    sf = jnp.sum(x * fp, axis=0) / (qn * nf)
    sb = jnp.sum(x * bp, axis=0) / (qn * nb)
    z = 10.0 * (sf - sb)
    pf = 1.0 / (1.0 + jnp.exp(-z))
    pb = 1.0 / (1.0 + jnp.exp(z))
    pf_ref[0] = pf
    pb_ref[0] = pb
    mf = (pf > tf).astype(jnp.float32)
    mb = (pb > tb).astype(jnp.float32)
    sumf = jnp.sum(jnp.sum(x * mf[None], axis=2, keepdims=True),
                   axis=1, keepdims=True)[None]
    sumb = jnp.sum(jnp.sum(x * mb[None], axis=2, keepdims=True),
                   axis=1, keepdims=True)[None]
    lane = lax.broadcasted_iota(jnp.int32, (1, 1, 1, 8), 3)
    crow = jnp.where(lane == 0, jnp.sum(mf),
                     jnp.where(lane == 1, jnp.sum(mb), 0.0))

    @pl.when(h == 0)
    def _():
        af_ref[...] = sumf
        ab_ref[...] = sumb
        cnt_ref[...] = crow

    @pl.when(h != 0)
    def _():
        af_ref[...] += sumf
        ab_ref[...] += sumb
        cnt_ref[...] += crow


def _tk1_body(pf_ref, pb_ref, wf_ref, wb_ref):
    hh, ww = pf_ref.shape[-2], pf_ref.shape[-1]
    flat = (lax.broadcasted_iota(jnp.int32, (hh, ww), 0) * ww
            + lax.broadcasted_iota(jnp.int32, (hh, ww), 1))

    def topw(p):
        x = p
        w = jnp.zeros_like(p)
        for _ in range(TOPK):
            m = jnp.max(x)
            fi = jnp.min(jnp.where(x == m, flat, hh * ww))
            hit = flat == fi
            w = w + hit.astype(jnp.float32)
            x = jnp.where(hit, -jnp.inf, x)
        return w

    wf_ref[0] = topw(pf_ref[0])
    wb_ref[0] = topw(pb_ref[0])


def _tk2_body(q_ref, wf_ref, wb_ref, tf_ref, tb_ref):
    h = pl.program_id(1)
    x = q_ref[0]
    wf = wf_ref[0]
    wb = wb_ref[0]
    sumf = jnp.sum(jnp.sum(x * wf[None], axis=2, keepdims=True),
                   axis=1, keepdims=True)[None]
    sumb = jnp.sum(jnp.sum(x * wb[None], axis=2, keepdims=True),
                   axis=1, keepdims=True)[None]

    @pl.when(h == 0)
    def _():
        tf_ref[...] = sumf
        tb_ref[...] = sumb

    @pl.when(h != 0)
    def _():
        tf_ref[...] += sumf
        tb_ref[...] += sumb


def _p2_body(fgp_ref, bgp_ref, q_ref, a_ref, d_ref, mm_ref):
    b = pl.program_id(0)
    h = pl.program_id(1)
    x = q_ref[0]
    fg = fgp_ref[0]
    bg = bgp_ref[0]
    qn = jnp.maximum(jnp.sqrt(jnp.sum(x * x, axis=0)), EPS)
    nf = jnp.maximum(jnp.sqrt(jnp.sum(fg * fg)), EPS)
    nb = jnp.maximum(jnp.sqrt(jnp.sum(bg * bg)), EPS)
    a = jnp.sum(x * fg, axis=0) / (qn * nf)
    d = jnp.sum(x * bg, axis=0) / (qn * nb)
    a_ref[0] = a
    d_ref[0] = d
    lane = lax.broadcasted_iota(jnp.int32, (1, 8), 1)
    row = jnp.where(lane == 0, jnp.min(a),
                    jnp.where(lane == 1, jnp.max(a),
                              jnp.where(lane == 2, jnp.min(d),
                                        jnp.where(lane == 3, jnp.max(d),
                                                  0.0))))

    @pl.when(jnp.logical_and(b == 0, h == 0))
    def _():
        mm_ref[...] = row

    @pl.when(jnp.logical_or(b != 0, h != 0))
    def _():
        cur = mm_ref[...]
        minl = jnp.logical_or(lane == 0, lane == 2)
        mm_ref[...] = jnp.where(minl, jnp.minimum(cur, row),
                                jnp.maximum(cur, row))


def _p3_body(mm_ref, q_ref, a_ref, d_ref, o_ref):
    x = q_ref[0]
    a = a_ref[0]
    d = d_ref[0]
    an = (a - mm_ref[0]) / (mm_ref[1] - mm_ref[0])
    dn = (d - mm_ref[2]) / (mm_ref[3] - mm_ref[2])
    o_ref[0] = x * (an + (1.0 - dn))[None]


def kernel(supp_fp, supp_bp, query_fea, tau):
    bs, C, H, W = query_fea.shape
    BH = 32
    nblk = H // BH
    f32 = jnp.float32
    ft = jax.nn.sigmoid(tau)
    th = jnp.stack([ft, 1.0 - ft]).astype(f32)

    pf, pb, af, ab, cnt = pl.pallas_call(
        _p1_body,
        grid=(bs, nblk),
        in_specs=[
            pl.BlockSpec(memory_space=pltpu.SMEM),
            pl.BlockSpec((1, C, 1, 1), lambda b, h: (b, 0, 0, 0)),
            pl.BlockSpec((1, C, 1, 1), lambda b, h: (b, 0, 0, 0)),
            pl.BlockSpec((1, C, BH, W), lambda b, h: (b, 0, h, 0)),
        ],
        out_specs=[
            pl.BlockSpec((1, BH, W), lambda b, h: (b, h, 0)),
            pl.BlockSpec((1, BH, W), lambda b, h: (b, h, 0)),
            pl.BlockSpec((1, C, 1, 1), lambda b, h: (b, 0, 0, 0)),
            pl.BlockSpec((1, C, 1, 1), lambda b, h: (b, 0, 0, 0)),
            pl.BlockSpec((1, 1, 1, 8), lambda b, h: (b, 0, 0, 0)),
        ],
        out_shape=[
            jax.ShapeDtypeStruct((bs, H, W), f32),
            jax.ShapeDtypeStruct((bs, H, W), f32),
            jax.ShapeDtypeStruct((bs, C, 1, 1), f32),
            jax.ShapeDtypeStruct((bs, C, 1, 1), f32),
            jax.ShapeDtypeStruct((bs, 1, 1, 8), f32),
        ],
    )(th, supp_fp, supp_bp, query_fea)

    cf = cnt[:, 0, 0, 0]
    cb = cnt[:, 0, 0, 1]
    need = jnp.logical_or(jnp.any(cf == 0), jnp.any(cb == 0))

    def topk_fn(_):
        wf, wb = pl.pallas_call(
            _tk1_body,
            grid=(bs,),
            in_specs=[
                pl.BlockSpec((1, H, W), lambda b: (b, 0, 0)),
                pl.BlockSpec((1, H, W), lambda b: (b, 0, 0)),
            ],
            out_specs=[
                pl.BlockSpec((1, H, W), lambda b: (b, 0, 0)),
                pl.BlockSpec((1, H, W), lambda b: (b, 0, 0)),
            ],
            out_shape=[
                jax.ShapeDtypeStruct((bs, H, W), f32),
                jax.ShapeDtypeStruct((bs, H, W), f32),
            ],
        )(pf, pb)
        tf, tb = pl.pallas_call(
            _tk2_body,
            grid=(bs, nblk),
            in_specs=[
                pl.BlockSpec((1, C, BH, W), lambda b, h: (b, 0, h, 0)),
                pl.BlockSpec((1, BH, W), lambda b, h: (b, h, 0)),
                pl.BlockSpec((1, BH, W), lambda b, h: (b, h, 0)),
            ],
            out_specs=[
                pl.BlockSpec((1, C, 1, 1), lambda b, h: (b, 0, 0, 0)),
                pl.BlockSpec((1, C, 1, 1), lambda b, h: (b, 0, 0, 0)),
            ],
            out_shape=[
                jax.ShapeDtypeStruct((bs, C, 1, 1), f32),
                jax.ShapeDtypeStruct((bs, C, 1, 1), f32),
            ],
        )(query_fea, wf, wb)
        return tf / TOPK, tb / TOPK

    def zeros_fn(_):
        return (jnp.zeros((bs, C, 1, 1), f32), jnp.zeros((bs, C, 1, 1), f32))

    tkf, tkb = lax.cond(need, topk_fn, zeros_fn, None)

    cf4 = cf[:, None, None, None]
    cb4 = cb[:, None, None, None]
    fgp = jnp.where(cf4 > 0, af / jnp.maximum(cf4, 1.0), tkf)
    bgp = jnp.where(cb4 > 0, ab / jnp.maximum(cb4, 1.0), tkb)

    a, d, mm = pl.pallas_call(
        _p2_body,
        grid=(bs, nblk),
        in_specs=[
            pl.BlockSpec((1, C, 1, 1), lambda b, h: (b, 0, 0, 0)),
            pl.BlockSpec((1, C, 1, 1), lambda b, h: (b, 0, 0, 0)),
            pl.BlockSpec((1, C, BH, W), lambda b, h: (b, 0, h, 0)),
        ],
        out_specs=[
            pl.BlockSpec((1, BH, W), lambda b, h: (b, h, 0)),
            pl.BlockSpec((1, BH, W), lambda b, h: (b, h, 0)),
            pl.BlockSpec((1, 8), lambda b, h: (0, 0)),
        ],
        out_shape=[
            jax.ShapeDtypeStruct((bs, H, W), f32),
            jax.ShapeDtypeStruct((bs, H, W), f32),
            jax.ShapeDtypeStruct((1, 8), f32),
        ],
    )(fgp, bgp, query_fea)

    qo = pl.pallas_call(
        _p3_body,
        grid=(bs, nblk),
        in_specs=[
            pl.BlockSpec(memory_space=pltpu.SMEM),
            pl.BlockSpec((1, C, BH, W), lambda b, h: (b, 0, h, 0)),
            pl.BlockSpec((1, BH, W), lambda b, h: (b, h, 0)),
            pl.BlockSpec((1, BH, W), lambda b, h: (b, h, 0)),
        ],
        out_specs=pl.BlockSpec((1, C, BH, W), lambda b, h: (b, 0, h, 0)),
        out_shape=jax.ShapeDtypeStruct((bs, C, H, W), f32),
    )(mm.reshape(8), query_fea, a, d)

    return (qo, fgp, bgp)


# speedup vs baseline: 2.2871x; 1.1243x over previous
import jax
import jax.numpy as jnp
from jax import lax
from jax.experimental import pallas as pl
from jax.experimental.pallas import tpu as pltpu

EPS = 1e-8
TOPK = 12


def _fold8(t):
    s = t.shape[-2]
    acc = t[..., 0:8, :]
    for i in range(8, s, 8):
        acc = acc + t[..., i:i + 8, :]
    return acc


def _p1_body(th_ref, w_ref, q_ref, pf_ref, iq_ref, af_ref, at_ref, cnt_ref,
             fs_scr, xs_scr, cf_scr):
    h = pl.program_id(1)
    last = pl.num_programs(1) - 1
    x = q_ref[0]
    wv = w_ref[0]
    tf = th_ref[0]
    s2 = jnp.sum(x * x, axis=0)
    dw = jnp.sum(x * wv, axis=0)
    iq = 1.0 / jnp.maximum(jnp.sqrt(s2), EPS)
    pf = 1.0 / (1.0 + jnp.exp(-(dw * iq)))
    pf_ref[0] = pf
    iq_ref[0] = iq
    mf = (pf > tf).astype(jnp.float32)
    pxm = _fold8(x * mf[None])
    pxs = _fold8(x)
    pcf = _fold8(mf)

    @pl.when(h == 0)
    def _():
        fs_scr[...] = pxm
        xs_scr[...] = pxs
        cf_scr[...] = pcf

    @pl.when(h != 0)
    def _():
        fs_scr[...] += pxm
        xs_scr[...] += pxs
        cf_scr[...] += pcf

    @pl.when(h == last)
    def _():
        red = lambda t: jnp.sum(jnp.sum(t, axis=2, keepdims=True),
                                axis=1, keepdims=True)[None]
        af_ref[...] = red(fs_scr[...])
        at_ref[...] = red(xs_scr[...])
        lane = lax.broadcasted_iota(jnp.int32, (1, 1, 1, 8), 3)
        cnt_ref[...] = jnp.where(lane == 0, jnp.sum(cf_scr[...]), 0.0)


def _tk1_body(pf_ref, wf_ref, wb_ref):
    hh, ww = pf_ref.shape[-2], pf_ref.shape[-1]
    flat = (lax.broadcasted_iota(jnp.int32, (hh, ww), 0) * ww
            + lax.broadcasted_iota(jnp.int32, (hh, ww), 1))

    def topw(p):
        x = p
        w = jnp.zeros_like(p)
        for _ in range(TOPK):
            m = jnp.max(x)
            fi = jnp.min(jnp.where(x == m, flat, hh * ww))
            hit = flat == fi
            w = w + hit.astype(jnp.float32)
            x = jnp.where(hit, -jnp.inf, x)
        return w

    p = pf_ref[0]
    wf_ref[0] = topw(p)
    wb_ref[0] = topw(1.0 - p)


def _tk2_body(q_ref, wf_ref, wb_ref, tf_ref, tb_ref):
    h = pl.program_id(1)
    x = q_ref[0]
    wf = wf_ref[0]
    wb = wb_ref[0]
    red = lambda t: jnp.sum(jnp.sum(t, axis=2, keepdims=True),
                            axis=1, keepdims=True)[None]
    sumf = red(x * wf[None])
    sumb = red(x * wb[None])

    @pl.when(h == 0)
    def _():
        tf_ref[...] = sumf
        tb_ref[...] = sumb

    @pl.when(h != 0)
    def _():
        tf_ref[...] += sumf
        tb_ref[...] += sumb


def _p2_body(fgn_ref, bgn_ref, q_ref, iq_ref, a_ref, d_ref, mm_ref,
             an_scr, ax_scr, dn_scr, dx_scr):
    b = pl.program_id(0)
    h = pl.program_id(1)
    first = jnp.logical_and(b == 0, h == 0)
    last = jnp.logical_and(b == pl.num_programs(0) - 1,
                           h == pl.num_programs(1) - 1)
    x = q_ref[0]
    fg = fgn_ref[0]
    bg = bgn_ref[0]
    iq = iq_ref[0]
    a = jnp.sum(x * fg, axis=0) * iq
    d = jnp.sum(x * bg, axis=0) * iq
    a_ref[0] = a
    d_ref[0] = d

    def fold(t, op):
        acc = t[0:8]
        for i in range(8, t.shape[0], 8):
            acc = op(acc, t[i:i + 8])
        return acc

    an = fold(a, jnp.minimum)
    ax = fold(a, jnp.maximum)
    dn = fold(d, jnp.minimum)
    dx = fold(d, jnp.maximum)

    @pl.when(first)
    def _():
        an_scr[...] = an
        ax_scr[...] = ax
        dn_scr[...] = dn
        dx_scr[...] = dx

    @pl.when(jnp.logical_not(first))
    def _():
        an_scr[...] = jnp.minimum(an_scr[...], an)
        ax_scr[...] = jnp.maximum(ax_scr[...], ax)
        dn_scr[...] = jnp.minimum(dn_scr[...], dn)
        dx_scr[...] = jnp.maximum(dx_scr[...], dx)

    @pl.when(last)
    def _():
        lane = lax.broadcasted_iota(jnp.int32, (1, 8), 1)
        row = jnp.where(lane == 0, jnp.min(an_scr[...]),
                        jnp.where(lane == 1, jnp.max(ax_scr[...]),
                                  jnp.where(lane == 2, jnp.min(dn_scr[...]),
                                            jnp.where(lane == 3,
                                                      jnp.max(dx_scr[...]),
                                                      0.0))))
        mm_ref[...] = row


def _p3_body(mm_ref, q_ref, a_ref, d_ref, o_ref):
    x = q_ref[0]
    a = a_ref[0]
    d = d_ref[0]
    an = (a - mm_ref[0]) / (mm_ref[1] - mm_ref[0])
    dn = (d - mm_ref[2]) / (mm_ref[3] - mm_ref[2])
    o_ref[0] = x * (an + (1.0 - dn))[None]


def kernel(supp_fp, supp_bp, query_fea, tau):
    bs, C, H, W = query_fea.shape
    hw = H * W
    BH = 32
    nblk = H // BH
    f32 = jnp.float32
    ft = jax.nn.sigmoid(tau)
    th = jnp.stack([ft, 1.0 - ft]).astype(f32)

    fp = supp_fp[:, :, 0, 0]
    bp = supp_bp[:, :, 0, 0]
    nf = jnp.maximum(jnp.sqrt(jnp.sum(fp * fp, axis=1)), EPS)[:, None]
    nb = jnp.maximum(jnp.sqrt(jnp.sum(bp * bp, axis=1)), EPS)[:, None]
    wvec = (10.0 * (fp / nf - bp / nb)).reshape(bs, C, 1, 1)

    pf, iq, af, at, cnt = pl.pallas_call(
        _p1_body,
        grid=(bs, nblk),
        in_specs=[
            pl.BlockSpec(memory_space=pltpu.SMEM),
            pl.BlockSpec((1, C, 1, 1), lambda b, h: (b, 0, 0, 0)),
            pl.BlockSpec((1, C, BH, W), lambda b, h: (b, 0, h, 0)),
        ],
        out_specs=[
            pl.BlockSpec((1, BH, W), lambda b, h: (b, h, 0)),
            pl.BlockSpec((1, BH, W), lambda b, h: (b, h, 0)),
            pl.BlockSpec((1, C, 1, 1), lambda b, h: (b, 0, 0, 0)),
            pl.BlockSpec((1, C, 1, 1), lambda b, h: (b, 0, 0, 0)),
            pl.BlockSpec((1, 1, 1, 8), lambda b, h: (b, 0, 0, 0)),
        ],
        out_shape=[
            jax.ShapeDtypeStruct((bs, H, W), f32),
            jax.ShapeDtypeStruct((bs, H, W), f32),
            jax.ShapeDtypeStruct((bs, C, 1, 1), f32),
            jax.ShapeDtypeStruct((bs, C, 1, 1), f32),
            jax.ShapeDtypeStruct((bs, 1, 1, 8), f32),
        ],
        scratch_shapes=[
            pltpu.VMEM((C, 8, W), f32),
            pltpu.VMEM((C, 8, W), f32),
            pltpu.VMEM((8, W), f32),
        ],
    )(th, wvec, query_fea)

    cf = cnt[:, 0, 0, 0]
    cb = hw - cf
    need = jnp.logical_or(jnp.any(cf == 0), jnp.any(cb == 0))

    def topk_fn(_):
        wf, wb = pl.pallas_call(
            _tk1_body,
            grid=(bs,),
            in_specs=[pl.BlockSpec((1, H, W), lambda b: (b, 0, 0))],
            out_specs=[
                pl.BlockSpec((1, H, W), lambda b: (b, 0, 0)),
                pl.BlockSpec((1, H, W), lambda b: (b, 0, 0)),
            ],
            out_shape=[
                jax.ShapeDtypeStruct((bs, H, W), f32),
                jax.ShapeDtypeStruct((bs, H, W), f32),
            ],
        )(pf)
        tf, tb = pl.pallas_call(
            _tk2_body,
            grid=(bs, nblk),
            in_specs=[
                pl.BlockSpec((1, C, BH, W), lambda b, h: (b, 0, h, 0)),
                pl.BlockSpec((1, BH, W), lambda b, h: (b, h, 0)),
                pl.BlockSpec((1, BH, W), lambda b, h: (b, h, 0)),
            ],
            out_specs=[
                pl.BlockSpec((1, C, 1, 1), lambda b, h: (b, 0, 0, 0)),
                pl.BlockSpec((1, C, 1, 1), lambda b, h: (b, 0, 0, 0)),
            ],
            out_shape=[
                jax.ShapeDtypeStruct((bs, C, 1, 1), f32),
                jax.ShapeDtypeStruct((bs, C, 1, 1), f32),
            ],
        )(query_fea, wf, wb)
        return tf / TOPK, tb / TOPK

    def zeros_fn(_):
        return (jnp.zeros((bs, C, 1, 1), f32), jnp.zeros((bs, C, 1, 1), f32))

    tkf, tkb = lax.cond(need, topk_fn, zeros_fn, None)

    cf4 = cf[:, None, None, None]
    cb4 = cb[:, None, None, None]
    ab = at - af
    fgp = jnp.where(cf4 > 0, af / jnp.maximum(cf4, 1.0), tkf)
    bgp = jnp.where(cb4 > 0, ab / jnp.maximum(cb4, 1.0), tkb)

    fgp2 = fgp[:, :, 0, 0]
    bgp2 = bgp[:, :, 0, 0]
    nfg = jnp.maximum(jnp.sqrt(jnp.sum(fgp2 * fgp2, axis=1)), EPS)[:, None]
    nbg = jnp.maximum(jnp.sqrt(jnp.sum(bgp2 * bgp2, axis=1)), EPS)[:, None]
    fgn = (fgp2 / nfg).reshape(bs, C, 1, 1)
    bgn = (bgp2 / nbg).reshape(bs, C, 1, 1)

    a, d, mm = pl.pallas_call(
        _p2_body,
        grid=(bs, nblk),
        in_specs=[
            pl.BlockSpec((1, C, 1, 1), lambda b, h: (b, 0, 0, 0)),
            pl.BlockSpec((1, C, 1, 1), lambda b, h: (b, 0, 0, 0)),
            pl.BlockSpec((1, C, BH, W), lambda b, h: (b, 0, h, 0)),
            pl.BlockSpec((1, BH, W), lambda b, h: (b, h, 0)),
        ],
        out_specs=[
            pl.BlockSpec((1, BH, W), lambda b, h: (b, h, 0)),
            pl.BlockSpec((1, BH, W), lambda b, h: (b, h, 0)),
            pl.BlockSpec((1, 8), lambda b, h: (0, 0)),
        ],
        out_shape=[
            jax.ShapeDtypeStruct((bs, H, W), f32),
            jax.ShapeDtypeStruct((bs, H, W), f32),
            jax.ShapeDtypeStruct((1, 8), f32),
        ],
        scratch_shapes=[
            pltpu.VMEM((8, W), f32),
            pltpu.VMEM((8, W), f32),
            pltpu.VMEM((8, W), f32),
            pltpu.VMEM((8, W), f32),
        ],
    )(fgn, bgn, query_fea, iq)

    qo = pl.pallas_call(
        _p3_body,
        grid=(bs, nblk),
        in_specs=[
            pl.BlockSpec(memory_space=pltpu.SMEM),
            pl.BlockSpec((1, C, BH, W), lambda b, h: (b, 0, h, 0)),
            pl.BlockSpec((1, BH, W), lambda b, h: (b, h, 0)),
            pl.BlockSpec((1, BH, W), lambda b, h: (b, h, 0)),
        ],
        out_specs=pl.BlockSpec((1, C, BH, W), lambda b, h: (b, 0, h, 0)),
        out_shape=jax.ShapeDtypeStruct((bs, C, H, W), f32),
    )(mm.reshape(8), query_fea, a, d)

    return (qo, fgp, bgp)


# speedup vs baseline: 2.8194x; 1.2327x over previous
import jax
import jax.numpy as jnp
from jax import lax
from jax.experimental import pallas as pl
from jax.experimental.pallas import tpu as pltpu

EPS = 1e-8
TOPK = 12


def _fold8(t):
    s = t.shape[-2]
    acc = t[..., 0:8, :]
    for i in range(8, s, 8):
        acc = acc + t[..., i:i + 8, :]
    return acc


def _p1_body(th_ref, w_ref, q_ref, pf_ref, iq_ref, af_ref, at_ref, cnt_ref,
             fs_scr, xs_scr, cf_scr):
    h = pl.program_id(1)
    last = pl.num_programs(1) - 1
    x = q_ref[0]
    wv = w_ref[0]
    tf = th_ref[0]
    s2 = jnp.sum(x * x, axis=0)
    dw = jnp.sum(x * wv, axis=0)
    iq = 1.0 / jnp.maximum(jnp.sqrt(s2), EPS)
    pf = 1.0 / (1.0 + jnp.exp(-(dw * iq)))
    pf_ref[0] = pf
    iq_ref[0] = iq
    mf = (pf > tf).astype(jnp.float32)
    pxm = _fold8(x * mf[None])
    pxs = _fold8(x)
    pcf = _fold8(mf)

    @pl.when(h == 0)
    def _():
        fs_scr[...] = pxm
        xs_scr[...] = pxs
        cf_scr[...] = pcf

    @pl.when(h != 0)
    def _():
        fs_scr[...] += pxm
        xs_scr[...] += pxs
        cf_scr[...] += pcf

    @pl.when(h == last)
    def _():
        red = lambda t: jnp.sum(jnp.sum(t, axis=2, keepdims=True),
                                axis=1, keepdims=True)[None]
        af_ref[...] = red(fs_scr[...])
        at_ref[...] = red(xs_scr[...])
        lane = lax.broadcasted_iota(jnp.int32, (1, 1, 1, 8), 3)
        cnt_ref[...] = jnp.where(lane == 0, jnp.sum(cf_scr[...]), 0.0)


def _tk1_body(pf_ref, wf_ref, wb_ref):
    hh, ww = pf_ref.shape[-2], pf_ref.shape[-1]
    flat = (lax.broadcasted_iota(jnp.int32, (hh, ww), 0) * ww
            + lax.broadcasted_iota(jnp.int32, (hh, ww), 1))

    def topw(p):
        x = p
        w = jnp.zeros_like(p)
        for _ in range(TOPK):
            m = jnp.max(x)
            fi = jnp.min(jnp.where(x == m, flat, hh * ww))
            hit = flat == fi
            w = w + hit.astype(jnp.float32)
            x = jnp.where(hit, -jnp.inf, x)
        return w

    p = pf_ref[0]
    wf_ref[0] = topw(p)
    wb_ref[0] = topw(1.0 - p)


def _tk2_body(q_ref, wf_ref, wb_ref, tf_ref, tb_ref):
    h = pl.program_id(1)
    x = q_ref[0]
    wf = wf_ref[0]
    wb = wb_ref[0]
    red = lambda t: jnp.sum(jnp.sum(t, axis=2, keepdims=True),
                            axis=1, keepdims=True)[None]
    sumf = red(x * wf[None])
    sumb = red(x * wb[None])

    @pl.when(h == 0)
    def _():
        tf_ref[...] = sumf
        tb_ref[...] = sumb

    @pl.when(h != 0)
    def _():
        tf_ref[...] += sumf
        tb_ref[...] += sumb


def _p2_body(fgn_ref, bgn_ref, q_ref, iq_ref, a_ref, d_ref, mm_ref,
             an_scr, ax_scr, dn_scr, dx_scr):
    b = pl.program_id(0)
    h = pl.program_id(1)
    first = jnp.logical_and(b == 0, h == 0)
    last = jnp.logical_and(b == pl.num_programs(0) - 1,
                           h == pl.num_programs(1) - 1)
    x = q_ref[0]
    fg = fgn_ref[0]
    bg = bgn_ref[0]
    iq = iq_ref[0]
    a = jnp.sum(x * fg, axis=0) * iq
    d = jnp.sum(x * bg, axis=0) * iq
    a_ref[0] = a
    d_ref[0] = d

    def fold(t, op):
        acc = t[0:8]
        for i in range(8, t.shape[0], 8):
            acc = op(acc, t[i:i + 8])
        return acc

    an = fold(a, jnp.minimum)
    ax = fold(a, jnp.maximum)
    dn = fold(d, jnp.minimum)
    dx = fold(d, jnp.maximum)

    @pl.when(first)
    def _():
        an_scr[...] = an
        ax_scr[...] = ax
        dn_scr[...] = dn
        dx_scr[...] = dx

    @pl.when(jnp.logical_not(first))
    def _():
        an_scr[...] = jnp.minimum(an_scr[...], an)
        ax_scr[...] = jnp.maximum(ax_scr[...], ax)
        dn_scr[...] = jnp.minimum(dn_scr[...], dn)
        dx_scr[...] = jnp.maximum(dx_scr[...], dx)

    @pl.when(last)
    def _():
        lane = lax.broadcasted_iota(jnp.int32, (1, 8), 1)
        row = jnp.where(lane == 0, jnp.min(an_scr[...]),
                        jnp.where(lane == 1, jnp.max(ax_scr[...]),
                                  jnp.where(lane == 2, jnp.min(dn_scr[...]),
                                            jnp.where(lane == 3,
                                                      jnp.max(dx_scr[...]),
                                                      0.0))))
        mm_ref[...] = row


def _p3_body(mm_ref, q_ref, a_ref, d_ref, o_ref):
    x = q_ref[0]
    a = a_ref[0]
    d = d_ref[0]
    an = (a - mm_ref[0]) / (mm_ref[1] - mm_ref[0])
    dn = (d - mm_ref[2]) / (mm_ref[3] - mm_ref[2])
    o_ref[0] = x * (an + (1.0 - dn))[None]


def kernel(supp_fp, supp_bp, query_fea, tau):
    bs, C, H, W = query_fea.shape
    hw = H * W
    BH = 64
    nblk = H // BH
    f32 = jnp.float32
    ft = jax.nn.sigmoid(tau)
    th = jnp.stack([ft, 1.0 - ft]).astype(f32)

    fp = supp_fp[:, :, 0, 0]
    bp = supp_bp[:, :, 0, 0]
    nf = jnp.maximum(jnp.sqrt(jnp.sum(fp * fp, axis=1)), EPS)[:, None]
    nb = jnp.maximum(jnp.sqrt(jnp.sum(bp * bp, axis=1)), EPS)[:, None]
    wvec = (10.0 * (fp / nf - bp / nb)).reshape(bs, C, 1, 1)

    pf, iq, af, at, cnt = pl.pallas_call(
        _p1_body,
        grid=(bs, nblk),
        in_specs=[
            pl.BlockSpec(memory_space=pltpu.SMEM),
            pl.BlockSpec((1, C, 1, 1), lambda b, h: (b, 0, 0, 0)),
            pl.BlockSpec((1, C, BH, W), lambda b, h: (b, 0, h, 0)),
        ],
        out_specs=[
            pl.BlockSpec((1, BH, W), lambda b, h: (b, h, 0)),
            pl.BlockSpec((1, BH, W), lambda b, h: (b, h, 0)),
            pl.BlockSpec((1, C, 1, 1), lambda b, h: (b, 0, 0, 0)),
            pl.BlockSpec((1, C, 1, 1), lambda b, h: (b, 0, 0, 0)),
            pl.BlockSpec((1, 1, 1, 8), lambda b, h: (b, 0, 0, 0)),
        ],
        out_shape=[
            jax.ShapeDtypeStruct((bs, H, W), f32),
            jax.ShapeDtypeStruct((bs, H, W), f32),
            jax.ShapeDtypeStruct((bs, C, 1, 1), f32),
            jax.ShapeDtypeStruct((bs, C, 1, 1), f32),
            jax.ShapeDtypeStruct((bs, 1, 1, 8), f32),
        ],
        scratch_shapes=[
            pltpu.VMEM((C, 8, W), f32),
            pltpu.VMEM((C, 8, W), f32),
            pltpu.VMEM((8, W), f32),
        ],
    )(th, wvec, query_fea)

    cf = cnt[:, 0, 0, 0]
    cb = hw - cf
    need = jnp.logical_or(jnp.any(cf == 0), jnp.any(cb == 0))

    def topk_fn(_):
        wf, wb = pl.pallas_call(
            _tk1_body,
            grid=(bs,),
            in_specs=[pl.BlockSpec((1, H, W), lambda b: (b, 0, 0))],
            out_specs=[
                pl.BlockSpec((1, H, W), lambda b: (b, 0, 0)),
                pl.BlockSpec((1, H, W), lambda b: (b, 0, 0)),
            ],
            out_shape=[
                jax.ShapeDtypeStruct((bs, H, W), f32),
                jax.ShapeDtypeStruct((bs, H, W), f32),
            ],
        )(pf)
        tf, tb = pl.pallas_call(
            _tk2_body,
            grid=(bs, nblk),
            in_specs=[
                pl.BlockSpec((1, C, BH, W), lambda b, h: (b, 0, h, 0)),
                pl.BlockSpec((1, BH, W), lambda b, h: (b, h, 0)),
                pl.BlockSpec((1, BH, W), lambda b, h: (b, h, 0)),
            ],
            out_specs=[
                pl.BlockSpec((1, C, 1, 1), lambda b, h: (b, 0, 0, 0)),
                pl.BlockSpec((1, C, 1, 1), lambda b, h: (b, 0, 0, 0)),
            ],
            out_shape=[
                jax.ShapeDtypeStruct((bs, C, 1, 1), f32),
                jax.ShapeDtypeStruct((bs, C, 1, 1), f32),
            ],
        )(query_fea, wf, wb)
        return tf / TOPK, tb / TOPK

    def zeros_fn(_):
        return (jnp.zeros((bs, C, 1, 1), f32), jnp.zeros((bs, C, 1, 1), f32))

    tkf, tkb = lax.cond(need, topk_fn, zeros_fn, None)

    cf4 = cf[:, None, None, None]
    cb4 = cb[:, None, None, None]
    ab = at - af
    fgp = jnp.where(cf4 > 0, af / jnp.maximum(cf4, 1.0), tkf)
    bgp = jnp.where(cb4 > 0, ab / jnp.maximum(cb4, 1.0), tkb)

    fgp2 = fgp[:, :, 0, 0]
    bgp2 = bgp[:, :, 0, 0]
    nfg = jnp.maximum(jnp.sqrt(jnp.sum(fgp2 * fgp2, axis=1)), EPS)[:, None]
    nbg = jnp.maximum(jnp.sqrt(jnp.sum(bgp2 * bgp2, axis=1)), EPS)[:, None]
    fgn = (fgp2 / nfg).reshape(bs, C, 1, 1)
    bgn = (bgp2 / nbg).reshape(bs, C, 1, 1)

    a, d, mm = pl.pallas_call(
        _p2_body,
        grid=(bs, nblk),
        in_specs=[
            pl.BlockSpec((1, C, 1, 1), lambda b, h: (b, 0, 0, 0)),
            pl.BlockSpec((1, C, 1, 1), lambda b, h: (b, 0, 0, 0)),
            pl.BlockSpec((1, C, BH, W), lambda b, h: (b, 0, h, 0)),
            pl.BlockSpec((1, BH, W), lambda b, h: (b, h, 0)),
        ],
        out_specs=[
            pl.BlockSpec((1, BH, W), lambda b, h: (b, h, 0)),
            pl.BlockSpec((1, BH, W), lambda b, h: (b, h, 0)),
            pl.BlockSpec((1, 8), lambda b, h: (0, 0)),
        ],
        out_shape=[
            jax.ShapeDtypeStruct((bs, H, W), f32),
            jax.ShapeDtypeStruct((bs, H, W), f32),
            jax.ShapeDtypeStruct((1, 8), f32),
        ],
        scratch_shapes=[
            pltpu.VMEM((8, W), f32),
            pltpu.VMEM((8, W), f32),
            pltpu.VMEM((8, W), f32),
            pltpu.VMEM((8, W), f32),
        ],
    )(fgn, bgn, query_fea, iq)

    qo = pl.pallas_call(
        _p3_body,
        grid=(bs, nblk),
        in_specs=[
            pl.BlockSpec(memory_space=pltpu.SMEM),
            pl.BlockSpec((1, C, BH, W), lambda b, h: (b, 0, h, 0)),
            pl.BlockSpec((1, BH, W), lambda b, h: (b, h, 0)),
            pl.BlockSpec((1, BH, W), lambda b, h: (b, h, 0)),
        ],
        out_specs=pl.BlockSpec((1, C, BH, W), lambda b, h: (b, 0, h, 0)),
        out_shape=jax.ShapeDtypeStruct((bs, C, H, W), f32),
    )(mm.reshape(8), query_fea, a, d)

    return (qo, fgp, bgp)


# speedup vs baseline: 3.2278x; 1.1449x over previous
import jax
import jax.numpy as jnp
from jax import lax
from jax.experimental import pallas as pl
from jax.experimental.pallas import tpu as pltpu

EPS = 1e-8
TOPK = 12


def _fold8(t):
    s = t.shape[-2]
    acc = t[..., 0:8, :]
    for i in range(8, s, 8):
        acc = acc + t[..., i:i + 8, :]
    return acc


def _p1_body(th_ref, w_ref, q_ref, pf_ref, iq_ref, af_ref, at_ref, cnt_ref,
             fs_scr, xs_scr, cf_scr):
    h = pl.program_id(1)
    last = pl.num_programs(1) - 1
    x = q_ref[0]
    wv = w_ref[0]
    tf = th_ref[0]
    s2 = jnp.sum(x * x, axis=0)
    dw = jnp.sum(x * wv, axis=0)
    iq = 1.0 / jnp.maximum(jnp.sqrt(s2), EPS)
    pf = 1.0 / (1.0 + jnp.exp(-(dw * iq)))
    pf_ref[0] = pf
    iq_ref[0] = iq
    mf = (pf > tf).astype(jnp.float32)
    pxm = _fold8(x * mf[None])
    pxs = _fold8(x)
    pcf = _fold8(mf)

    @pl.when(h == 0)
    def _():
        fs_scr[...] = pxm
        xs_scr[...] = pxs
        cf_scr[...] = pcf

    @pl.when(h != 0)
    def _():
        fs_scr[...] += pxm
        xs_scr[...] += pxs
        cf_scr[...] += pcf

    @pl.when(h == last)
    def _():
        red = lambda t: jnp.sum(jnp.sum(t, axis=2, keepdims=True),
                                axis=1, keepdims=True)[None]
        af_ref[...] = red(fs_scr[...])
        at_ref[...] = red(xs_scr[...])
        lane = lax.broadcasted_iota(jnp.int32, (1, 1, 1, 8), 3)
        cnt_ref[...] = jnp.where(lane == 0, jnp.sum(cf_scr[...]), 0.0)


def _tk1_body(pf_ref, wf_ref, wb_ref):
    hh, ww = pf_ref.shape[-2], pf_ref.shape[-1]
    flat = (lax.broadcasted_iota(jnp.int32, (hh, ww), 0) * ww
            + lax.broadcasted_iota(jnp.int32, (hh, ww), 1))

    def topw(p):
        x = p
        w = jnp.zeros_like(p)
        for _ in range(TOPK):
            m = jnp.max(x)
            fi = jnp.min(jnp.where(x == m, flat, hh * ww))
            hit = flat == fi
            w = w + hit.astype(jnp.float32)
            x = jnp.where(hit, -jnp.inf, x)
        return w

    p = pf_ref[0]
    wf_ref[0] = topw(p)
    wb_ref[0] = topw(1.0 - p)


def _tk2_body(q_ref, wf_ref, wb_ref, tf_ref, tb_ref):
    h = pl.program_id(1)
    x = q_ref[0]
    wf = wf_ref[0]
    wb = wb_ref[0]
    red = lambda t: jnp.sum(jnp.sum(t, axis=2, keepdims=True),
                            axis=1, keepdims=True)[None]
    sumf = red(x * wf[None])
    sumb = red(x * wb[None])

    @pl.when(h == 0)
    def _():
        tf_ref[...] = sumf
        tb_ref[...] = sumb

    @pl.when(h != 0)
    def _():
        tf_ref[...] += sumf
        tb_ref[...] += sumb


def _p2_body(fgn_ref, bgn_ref, q_ref, iq_ref, a_ref, d_ref, mm_ref,
             an_scr, ax_scr, dn_scr, dx_scr):
    b = pl.program_id(0)
    h = pl.program_id(1)
    first = jnp.logical_and(b == 0, h == 0)
    last = jnp.logical_and(b == pl.num_programs(0) - 1,
                           h == pl.num_programs(1) - 1)
    x = q_ref[0]
    fg = fgn_ref[0]
    bg = bgn_ref[0]
    iq = iq_ref[0]
    a = jnp.sum(x * fg, axis=0) * iq
    d = jnp.sum(x * bg, axis=0) * iq
    a_ref[0] = a
    d_ref[0] = d

    def fold(t, op):
        acc = t[0:8]
        for i in range(8, t.shape[0], 8):
            acc = op(acc, t[i:i + 8])
        return acc

    an = fold(a, jnp.minimum)
    ax = fold(a, jnp.maximum)
    dn = fold(d, jnp.minimum)
    dx = fold(d, jnp.maximum)

    @pl.when(first)
    def _():
        an_scr[...] = an
        ax_scr[...] = ax
        dn_scr[...] = dn
        dx_scr[...] = dx

    @pl.when(jnp.logical_not(first))
    def _():
        an_scr[...] = jnp.minimum(an_scr[...], an)
        ax_scr[...] = jnp.maximum(ax_scr[...], ax)
        dn_scr[...] = jnp.minimum(dn_scr[...], dn)
        dx_scr[...] = jnp.maximum(dx_scr[...], dx)

    @pl.when(last)
    def _():
        lane = lax.broadcasted_iota(jnp.int32, (1, 8), 1)
        row = jnp.where(lane == 0, jnp.min(an_scr[...]),
                        jnp.where(lane == 1, jnp.max(ax_scr[...]),
                                  jnp.where(lane == 2, jnp.min(dn_scr[...]),
                                            jnp.where(lane == 3,
                                                      jnp.max(dx_scr[...]),
                                                      0.0))))
        mm_ref[...] = row


def _p3_body(mm_ref, q_ref, a_ref, d_ref, o_ref):
    x = q_ref[0]
    a = a_ref[0]
    d = d_ref[0]
    an = (a - mm_ref[0]) / (mm_ref[1] - mm_ref[0])
    dn = (d - mm_ref[2]) / (mm_ref[3] - mm_ref[2])
    o_ref[0] = x * (an + (1.0 - dn))[None]


def kernel(supp_fp, supp_bp, query_fea, tau):
    bs, C, H, W = query_fea.shape
    hw = H * W
    BH = 128
    nblk = H // BH
    f32 = jnp.float32
    ft = jax.nn.sigmoid(tau)
    th = jnp.stack([ft, 1.0 - ft]).astype(f32)

    fp = supp_fp[:, :, 0, 0]
    bp = supp_bp[:, :, 0, 0]
    nf = jnp.maximum(jnp.sqrt(jnp.sum(fp * fp, axis=1)), EPS)[:, None]
    nb = jnp.maximum(jnp.sqrt(jnp.sum(bp * bp, axis=1)), EPS)[:, None]
    wvec = (10.0 * (fp / nf - bp / nb)).reshape(bs, C, 1, 1)

    pf, iq, af, at, cnt = pl.pallas_call(
        _p1_body,
        grid=(bs, nblk),
        in_specs=[
            pl.BlockSpec(memory_space=pltpu.SMEM),
            pl.BlockSpec((1, C, 1, 1), lambda b, h: (b, 0, 0, 0)),
            pl.BlockSpec((1, C, BH, W), lambda b, h: (b, 0, h, 0)),
        ],
        out_specs=[
            pl.BlockSpec((1, BH, W), lambda b, h: (b, h, 0)),
            pl.BlockSpec((1, BH, W), lambda b, h: (b, h, 0)),
            pl.BlockSpec((1, C, 1, 1), lambda b, h: (b, 0, 0, 0)),
            pl.BlockSpec((1, C, 1, 1), lambda b, h: (b, 0, 0, 0)),
            pl.BlockSpec((1, 1, 1, 8), lambda b, h: (b, 0, 0, 0)),
        ],
        out_shape=[
            jax.ShapeDtypeStruct((bs, H, W), f32),
            jax.ShapeDtypeStruct((bs, H, W), f32),
            jax.ShapeDtypeStruct((bs, C, 1, 1), f32),
            jax.ShapeDtypeStruct((bs, C, 1, 1), f32),
            jax.ShapeDtypeStruct((bs, 1, 1, 8), f32),
        ],
        scratch_shapes=[
            pltpu.VMEM((C, 8, W), f32),
            pltpu.VMEM((C, 8, W), f32),
            pltpu.VMEM((8, W), f32),
        ],
    )(th, wvec, query_fea)

    cf = cnt[:, 0, 0, 0]
    cb = hw - cf
    need = jnp.logical_or(jnp.any(cf == 0), jnp.any(cb == 0))

    def topk_fn(_):
        wf, wb = pl.pallas_call(
            _tk1_body,
            grid=(bs,),
            in_specs=[pl.BlockSpec((1, H, W), lambda b: (b, 0, 0))],
            out_specs=[
                pl.BlockSpec((1, H, W), lambda b: (b, 0, 0)),
                pl.BlockSpec((1, H, W), lambda b: (b, 0, 0)),
            ],
            out_shape=[
                jax.ShapeDtypeStruct((bs, H, W), f32),
                jax.ShapeDtypeStruct((bs, H, W), f32),
            ],
        )(pf)
        tf, tb = pl.pallas_call(
            _tk2_body,
            grid=(bs, nblk),
            in_specs=[
                pl.BlockSpec((1, C, BH, W), lambda b, h: (b, 0, h, 0)),
                pl.BlockSpec((1, BH, W), lambda b, h: (b, h, 0)),
                pl.BlockSpec((1, BH, W), lambda b, h: (b, h, 0)),
            ],
            out_specs=[
                pl.BlockSpec((1, C, 1, 1), lambda b, h: (b, 0, 0, 0)),
                pl.BlockSpec((1, C, 1, 1), lambda b, h: (b, 0, 0, 0)),
            ],
            out_shape=[
                jax.ShapeDtypeStruct((bs, C, 1, 1), f32),
                jax.ShapeDtypeStruct((bs, C, 1, 1), f32),
            ],
        )(query_fea, wf, wb)
        return tf / TOPK, tb / TOPK

    def zeros_fn(_):
        return (jnp.zeros((bs, C, 1, 1), f32), jnp.zeros((bs, C, 1, 1), f32))

    tkf, tkb = lax.cond(need, topk_fn, zeros_fn, None)

    cf4 = cf[:, None, None, None]
    cb4 = cb[:, None, None, None]
    ab = at - af
    fgp = jnp.where(cf4 > 0, af / jnp.maximum(cf4, 1.0), tkf)
    bgp = jnp.where(cb4 > 0, ab / jnp.maximum(cb4, 1.0), tkb)

    fgp2 = fgp[:, :, 0, 0]
    bgp2 = bgp[:, :, 0, 0]
    nfg = jnp.maximum(jnp.sqrt(jnp.sum(fgp2 * fgp2, axis=1)), EPS)[:, None]
    nbg = jnp.maximum(jnp.sqrt(jnp.sum(bgp2 * bgp2, axis=1)), EPS)[:, None]
    fgn = (fgp2 / nfg).reshape(bs, C, 1, 1)
    bgn = (bgp2 / nbg).reshape(bs, C, 1, 1)

    a, d, mm = pl.pallas_call(
        _p2_body,
        grid=(bs, nblk),
        in_specs=[
            pl.BlockSpec((1, C, 1, 1), lambda b, h: (b, 0, 0, 0)),
            pl.BlockSpec((1, C, 1, 1), lambda b, h: (b, 0, 0, 0)),
            pl.BlockSpec((1, C, BH, W), lambda b, h: (b, 0, h, 0)),
            pl.BlockSpec((1, BH, W), lambda b, h: (b, h, 0)),
        ],
        out_specs=[
            pl.BlockSpec((1, BH, W), lambda b, h: (b, h, 0)),
            pl.BlockSpec((1, BH, W), lambda b, h: (b, h, 0)),
            pl.BlockSpec((1, 8), lambda b, h: (0, 0)),
        ],
        out_shape=[
            jax.ShapeDtypeStruct((bs, H, W), f32),
            jax.ShapeDtypeStruct((bs, H, W), f32),
            jax.ShapeDtypeStruct((1, 8), f32),
        ],
        scratch_shapes=[
            pltpu.VMEM((8, W), f32),
            pltpu.VMEM((8, W), f32),
            pltpu.VMEM((8, W), f32),
            pltpu.VMEM((8, W), f32),
        ],
    )(fgn, bgn, query_fea, iq)

    qo = pl.pallas_call(
        _p3_body,
        grid=(bs, nblk),
        in_specs=[
            pl.BlockSpec(memory_space=pltpu.SMEM),
            pl.BlockSpec((1, C, BH, W), lambda b, h: (b, 0, h, 0)),
            pl.BlockSpec((1, BH, W), lambda b, h: (b, h, 0)),
            pl.BlockSpec((1, BH, W), lambda b, h: (b, h, 0)),
        ],
        out_specs=pl.BlockSpec((1, C, BH, W), lambda b, h: (b, 0, h, 0)),
        out_shape=jax.ShapeDtypeStruct((bs, C, H, W), f32),
    )(mm.reshape(8), query_fea, a, d)

    return (qo, fgp, bgp)


# speedup vs baseline: 3.6790x; 1.1398x over previous
import jax
import jax.numpy as jnp
from jax import lax
from jax.experimental import pallas as pl
from jax.experimental.pallas import tpu as pltpu

EPS = 1e-8
TOPK = 12


def _fold8(t):
    s = t.shape[-2]
    acc = t[..., 0:8, :]
    for i in range(8, s, 8):
        acc = acc + t[..., i:i + 8, :]
    return acc


def _red(t):
    return jnp.sum(jnp.sum(t, axis=2, keepdims=True), axis=1,
                   keepdims=True)[None]


def _topw(p, k):
    hh, ww = p.shape
    flat = (lax.broadcasted_iota(jnp.int32, (hh, ww), 0) * ww
            + lax.broadcasted_iota(jnp.int32, (hh, ww), 1))
    x = p
    w = jnp.zeros_like(p)
    for _ in range(k):
        m = jnp.max(x)
        fi = jnp.min(jnp.where(x == m, flat, hh * ww))
        hit = flat == fi
        w = w + hit.astype(jnp.float32)
        x = jnp.where(hit, -jnp.inf, x)
    return w


def _p12_body(th_ref, w_ref, q_ref, a_ref, d_ref, mm_ref, fgp_ref, bgp_ref,
              an_scr, ax_scr, dn_scr, dx_scr):
    b = pl.program_id(0)
    hw = q_ref.shape[2] * q_ref.shape[3]
    x = q_ref[0]
    wv = w_ref[0]
    tf = th_ref[0]
    s2 = jnp.sum(x * x, axis=0)
    dw = jnp.sum(x * wv, axis=0)
    iq = 1.0 / jnp.maximum(jnp.sqrt(s2), EPS)
    pf = 1.0 / (1.0 + jnp.exp(-(dw * iq)))
    mf = (pf > tf).astype(jnp.float32)
    fsum = _red(_fold8(x * mf[None]))
    tsum = _red(_fold8(x))
    cf = jnp.sum(_fold8(mf))
    cb = hw - cf

    fgp = lax.cond(cf > 0,
                   lambda: fsum / cf,
                   lambda: _red(x * _topw(pf, TOPK)[None]) / TOPK)
    bgp = lax.cond(cb > 0,
                   lambda: (tsum - fsum) / cb,
                   lambda: _red(x * _topw(1.0 - pf, TOPK)[None]) / TOPK)
    fgp_ref[...] = fgp
    bgp_ref[...] = bgp

    fgn = fgp[0] / jnp.maximum(jnp.sqrt(jnp.sum(fgp * fgp)), EPS)
    bgn = bgp[0] / jnp.maximum(jnp.sqrt(jnp.sum(bgp * bgp)), EPS)
    a = jnp.sum(x * fgn, axis=0) * iq
    d = jnp.sum(x * bgn, axis=0) * iq
    a_ref[0] = a
    d_ref[0] = d

    def fold(t, op):
        acc = t[0:8]
        for i in range(8, t.shape[0], 8):
            acc = op(acc, t[i:i + 8])
        return acc

    an = fold(a, jnp.minimum)
    ax = fold(a, jnp.maximum)
    dn = fold(d, jnp.minimum)
    dx = fold(d, jnp.maximum)

    @pl.when(b == 0)
    def _():
        an_scr[...] = an
        ax_scr[...] = ax
        dn_scr[...] = dn
        dx_scr[...] = dx

    @pl.when(b != 0)
    def _():
        an_scr[...] = jnp.minimum(an_scr[...], an)
        ax_scr[...] = jnp.maximum(ax_scr[...], ax)
        dn_scr[...] = jnp.minimum(dn_scr[...], dn)
        dx_scr[...] = jnp.maximum(dx_scr[...], dx)

    @pl.when(b == pl.num_programs(0) - 1)
    def _():
        lane = lax.broadcasted_iota(jnp.int32, (1, 8), 1)
        row = jnp.where(lane == 0, jnp.min(an_scr[...]),
                        jnp.where(lane == 1, jnp.max(ax_scr[...]),
                                  jnp.where(lane == 2, jnp.min(dn_scr[...]),
                                            jnp.where(lane == 3,
                                                      jnp.max(dx_scr[...]),
                                                      0.0))))
        mm_ref[...] = row


def _p3_body(mm_ref, q_ref, a_ref, d_ref, o_ref):
    x = q_ref[0]
    a = a_ref[0]
    d = d_ref[0]
    an = (a - mm_ref[0]) / (mm_ref[1] - mm_ref[0])
    dn = (d - mm_ref[2]) / (mm_ref[3] - mm_ref[2])
    o_ref[0] = x * (an + (1.0 - dn))[None]


def kernel(supp_fp, supp_bp, query_fea, tau):
    bs, C, H, W = query_fea.shape
    f32 = jnp.float32
    ft = jax.nn.sigmoid(tau)
    th = jnp.stack([ft, 1.0 - ft]).astype(f32)

    fp = supp_fp[:, :, 0, 0]
    bp = supp_bp[:, :, 0, 0]
    nf = jnp.maximum(jnp.sqrt(jnp.sum(fp * fp, axis=1)), EPS)[:, None]
    nb = jnp.maximum(jnp.sqrt(jnp.sum(bp * bp, axis=1)), EPS)[:, None]
    wvec = (10.0 * (fp / nf - bp / nb)).reshape(bs, C, 1, 1)

    a, d, mm, fgp, bgp = pl.pallas_call(
        _p12_body,
        grid=(bs,),
        in_specs=[
            pl.BlockSpec(memory_space=pltpu.SMEM),
            pl.BlockSpec((1, C, 1, 1), lambda b: (b, 0, 0, 0)),
            pl.BlockSpec((1, C, H, W), lambda b: (b, 0, 0, 0)),
        ],
        out_specs=[
            pl.BlockSpec((1, H, W), lambda b: (b, 0, 0)),
            pl.BlockSpec((1, H, W), lambda b: (b, 0, 0)),
            pl.BlockSpec((1, 8), lambda b: (0, 0)),
            pl.BlockSpec((1, C, 1, 1), lambda b: (b, 0, 0, 0)),
            pl.BlockSpec((1, C, 1, 1), lambda b: (b, 0, 0, 0)),
        ],
        out_shape=[
            jax.ShapeDtypeStruct((bs, H, W), f32),
            jax.ShapeDtypeStruct((bs, H, W), f32),
            jax.ShapeDtypeStruct((1, 8), f32),
            jax.ShapeDtypeStruct((bs, C, 1, 1), f32),
            jax.ShapeDtypeStruct((bs, C, 1, 1), f32),
        ],
        scratch_shapes=[
            pltpu.VMEM((8, W), f32),
            pltpu.VMEM((8, W), f32),
            pltpu.VMEM((8, W), f32),
            pltpu.VMEM((8, W), f32),
        ],
    )(th, wvec, query_fea)

    qo = pl.pallas_call(
        _p3_body,
        grid=(bs,),
        in_specs=[
            pl.BlockSpec(memory_space=pltpu.SMEM),
            pl.BlockSpec((1, C, H, W), lambda b: (b, 0, 0, 0)),
            pl.BlockSpec((1, H, W), lambda b: (b, 0, 0)),
            pl.BlockSpec((1, H, W), lambda b: (b, 0, 0)),
        ],
        out_specs=pl.BlockSpec((1, C, H, W), lambda b: (b, 0, 0, 0)),
        out_shape=jax.ShapeDtypeStruct((bs, C, H, W), f32),
    )(mm.reshape(8), query_fea, a, d)

    return (qo, fgp, bgp)


# speedup vs baseline: 4.1222x; 1.1205x over previous
import jax
import jax.numpy as jnp
from jax import lax
from jax.experimental import pallas as pl
from jax.experimental.pallas import tpu as pltpu

EPS = 1e-8
TOPK = 12
SB = 32


def _fold8(t):
    s = t.shape[-2]
    acc = t[0:8, :]
    for i in range(8, s, 8):
        acc = acc + t[i:i + 8, :]
    return acc


def _red(t):
    return jnp.sum(jnp.sum(t, axis=2, keepdims=True), axis=1,
                   keepdims=True)[None]


def _topw(p, k):
    hh, ww = p.shape
    flat = (lax.broadcasted_iota(jnp.int32, (hh, ww), 0) * ww
            + lax.broadcasted_iota(jnp.int32, (hh, ww), 1))
    x = p
    w = jnp.zeros_like(p)
    for _ in range(k):
        m = jnp.max(x)
        fi = jnp.min(jnp.where(x == m, flat, hh * ww))
        hit = flat == fi
        w = w + hit.astype(jnp.float32)
        x = jnp.where(hit, -jnp.inf, x)
    return w


def _p12_body(th_ref, w_ref, q_ref, a_ref, d_ref, mm_ref, fgp_ref, bgp_ref,
              pf_scr, iq_scr, fs_scr, ts_scr, an_scr, ax_scr, dn_scr, dx_scr):
    b = pl.program_id(0)
    C, H, W = q_ref.shape[1], q_ref.shape[2], q_ref.shape[3]
    hw = H * W
    wv = w_ref[0]
    tf = th_ref[0]

    for hb in range(0, H, SB):
        xs = q_ref[0, :, hb:hb + SB, :]
        s2 = jnp.sum(xs * xs, axis=0)
        dw = jnp.sum(xs * wv, axis=0)
        iq = 1.0 / jnp.maximum(jnp.sqrt(s2), EPS)
        pf_scr[hb:hb + SB, :] = 1.0 / (1.0 + jnp.exp(-(dw * iq)))
        iq_scr[hb:hb + SB, :] = iq

    pf = pf_scr[...]
    mff = (pf > tf).astype(jnp.float32)
    cf = jnp.sum(_fold8(mff))
    cb = hw - cf

    for c in range(C):
        xc = q_ref[0, c]
        fs_scr[c] = _fold8(xc * mff)
        ts_scr[c] = _fold8(xc)
    fsum = _red(fs_scr[...])
    tsum = _red(ts_scr[...])

    fgp = lax.cond(cf > 0,
                   lambda: fsum / cf,
                   lambda: _red(q_ref[0] * _topw(pf, TOPK)[None]) / TOPK)
    bgp = lax.cond(cb > 0,
                   lambda: (tsum - fsum) / cb,
                   lambda: _red(q_ref[0] * _topw(1.0 - pf, TOPK)[None])
                   / TOPK)
    fgp_ref[...] = fgp
    bgp_ref[...] = bgp

    fgn = fgp[0] / jnp.maximum(jnp.sqrt(jnp.sum(fgp * fgp)), EPS)
    bgn = bgp[0] / jnp.maximum(jnp.sqrt(jnp.sum(bgp * bgp)), EPS)

    an = ax = dn = dx = None
    for hb in range(0, H, SB):
        xs = q_ref[0, :, hb:hb + SB, :]
        iq = iq_scr[hb:hb + SB, :]
        a = jnp.sum(xs * fgn, axis=0) * iq
        d = jnp.sum(xs * bgn, axis=0) * iq
        a_ref[0, hb:hb + SB, :] = a
        d_ref[0, hb:hb + SB, :] = d
        na = a[0:8, :]
        nd = d[0:8, :]
        ma = a[0:8, :]
        md = d[0:8, :]
        for i in range(8, SB, 8):
            na = jnp.minimum(na, a[i:i + 8, :])
            nd = jnp.minimum(nd, d[i:i + 8, :])
            ma = jnp.maximum(ma, a[i:i + 8, :])
            md = jnp.maximum(md, d[i:i + 8, :])
        an = na if an is None else jnp.minimum(an, na)
        dn = nd if dn is None else jnp.minimum(dn, nd)
        ax = ma if ax is None else jnp.maximum(ax, ma)
        dx = md if dx is None else jnp.maximum(dx, md)

    @pl.when(b == 0)
    def _():
        an_scr[...] = an
        ax_scr[...] = ax
        dn_scr[...] = dn
        dx_scr[...] = dx

    @pl.when(b != 0)
    def _():
        an_scr[...] = jnp.minimum(an_scr[...], an)
        ax_scr[...] = jnp.maximum(ax_scr[...], ax)
        dn_scr[...] = jnp.minimum(dn_scr[...], dn)
        dx_scr[...] = jnp.maximum(dx_scr[...], dx)

    @pl.when(b == pl.num_programs(0) - 1)
    def _():
        lane = lax.broadcasted_iota(jnp.int32, (1, 8), 1)
        row = jnp.where(lane == 0, jnp.min(an_scr[...]),
                        jnp.where(lane == 1, jnp.max(ax_scr[...]),
                                  jnp.where(lane == 2, jnp.min(dn_scr[...]),
                                            jnp.where(lane == 3,
                                                      jnp.max(dx_scr[...]),
                                                      0.0))))
        mm_ref[...] = row


def _p3_body(mm_ref, q_ref, a_ref, d_ref, o_ref):
    x = q_ref[0]
    a = a_ref[0]
    d = d_ref[0]
    an = (a - mm_ref[0]) / (mm_ref[1] - mm_ref[0])
    dn = (d - mm_ref[2]) / (mm_ref[3] - mm_ref[2])
    o_ref[0] = x * (an + (1.0 - dn))[None]


def kernel(supp_fp, supp_bp, query_fea, tau):
    bs, C, H, W = query_fea.shape
    f32 = jnp.float32
    ft = jax.nn.sigmoid(tau)
    th = jnp.stack([ft, 1.0 - ft]).astype(f32)

    fp = supp_fp[:, :, 0, 0]
    bp = supp_bp[:, :, 0, 0]
    nf = jnp.maximum(jnp.sqrt(jnp.sum(fp * fp, axis=1)), EPS)[:, None]
    nb = jnp.maximum(jnp.sqrt(jnp.sum(bp * bp, axis=1)), EPS)[:, None]
    wvec = (10.0 * (fp / nf - bp / nb)).reshape(bs, C, 1, 1)

    a, d, mm, fgp, bgp = pl.pallas_call(
        _p12_body,
        grid=(bs,),
        in_specs=[
            pl.BlockSpec(memory_space=pltpu.SMEM),
            pl.BlockSpec((1, C, 1, 1), lambda b: (b, 0, 0, 0)),
            pl.BlockSpec((1, C, H, W), lambda b: (b, 0, 0, 0)),
        ],
        out_specs=[
            pl.BlockSpec((1, H, W), lambda b: (b, 0, 0)),
            pl.BlockSpec((1, H, W), lambda b: (b, 0, 0)),
            pl.BlockSpec((1, 8), lambda b: (0, 0)),
            pl.BlockSpec((1, C, 1, 1), lambda b: (b, 0, 0, 0)),
            pl.BlockSpec((1, C, 1, 1), lambda b: (b, 0, 0, 0)),
        ],
        out_shape=[
            jax.ShapeDtypeStruct((bs, H, W), f32),
            jax.ShapeDtypeStruct((bs, H, W), f32),
            jax.ShapeDtypeStruct((1, 8), f32),
            jax.ShapeDtypeStruct((bs, C, 1, 1), f32),
            jax.ShapeDtypeStruct((bs, C, 1, 1), f32),
        ],
        scratch_shapes=[
            pltpu.VMEM((H, W), f32),
            pltpu.VMEM((H, W), f32),
            pltpu.VMEM((C, 8, W), f32),
            pltpu.VMEM((C, 8, W), f32),
            pltpu.VMEM((8, W), f32),
            pltpu.VMEM((8, W), f32),
            pltpu.VMEM((8, W), f32),
            pltpu.VMEM((8, W), f32),
        ],
    )(th, wvec, query_fea)

    qo = pl.pallas_call(
        _p3_body,
        grid=(bs,),
        in_specs=[
            pl.BlockSpec(memory_space=pltpu.SMEM),
            pl.BlockSpec((1, C, H, W), lambda b: (b, 0, 0, 0)),
            pl.BlockSpec((1, H, W), lambda b: (b, 0, 0)),
            pl.BlockSpec((1, H, W), lambda b: (b, 0, 0)),
        ],
        out_specs=pl.BlockSpec((1, C, H, W), lambda b: (b, 0, 0, 0)),
        out_shape=jax.ShapeDtypeStruct((bs, C, H, W), f32),
    )(mm.reshape(8), query_fea, a, d)

    return (qo, fgp, bgp)
